# baseline (speedup 1.0000x reference)
import jax
import jax.numpy as jnp
from jax import lax
from jax.experimental import pallas as pl
from jax.experimental.pallas import tpu as pltpu, tpu_sc as plsc

_VOCAB = 1000000
_EMB = 32
_B = 4096
_L = 50
_NC = 2
_NS = 16
_NW = _NC * _NS
_TOTAL = _B * _L
_PER_W = _TOTAL // _NW
_CHUNK = 128
_NCH = _PER_W // _CHUNK
_VROWS = _VOCAB // 4
_GRP = _CHUNK // 16

_mesh = plsc.VectorSubcoreMesh(
    core_axis_name="c", subcore_axis_name="s", num_cores=_NC, num_subcores=_NS
)


def _gather_body(vrow_hbm, coloff_hbm, table_hbm, out_hbm,
                 vrow_v, coloff_v, big_v, out_v, gsem0, gsem1):
    wid = lax.axis_index("s") * _NC + lax.axis_index("c")
    base = wid * _PER_W
    pltpu.sync_copy(vrow_hbm.at[wid], vrow_v)
    pltpu.sync_copy(coloff_hbm.at[wid], coloff_v)

    iota = lax.iota(jnp.int32, 16)
    colbase = iota * 32

    def _fire(j, b, sem):
        pltpu.async_copy(table_hbm.at[vrow_v.at[j]], big_v.at[b], sem)

    def _drain(b, sem):
        pltpu.make_async_copy(
            table_hbm.at[pl.ds(0, _CHUNK)], big_v.at[b], sem
        ).wait()

    def _extract_write(j, b):
        for g in range(_GRP):
            row16 = iota + g * 16
            col16 = coloff_v[j, pl.ds(g * 16, 16)]
            for c in range(_EMB):
                val = plsc.load_gather(big_v.at[b], [row16, col16 + c])
                plsc.store_scatter(out_v, [colbase + (g * 512 + c)], val)
        pltpu.sync_copy(
            out_v, out_hbm.at[pl.ds((base + j * _CHUNK) * _EMB, _CHUNK * _EMB)]
        )

    _fire(0, 0, gsem0)

    @pl.loop(0, _NCH, step=2)
    def _loop(j0):
        _fire(j0 + 1, 1, gsem1)
        _drain(0, gsem0)
        _extract_write(j0, 0)

        @pl.when(j0 + 2 < _NCH)
        def _():
            _fire(j0 + 2, 0, gsem0)

        _drain(1, gsem1)
        _extract_write(j0 + 1, 1)


_gather = pl.kernel(
    _gather_body,
    out_type=jax.ShapeDtypeStruct((_TOTAL * _EMB,), jnp.float32),
    mesh=_mesh,
    scratch_types=[
        pltpu.VMEM((_NCH, _CHUNK), jnp.int32),
        pltpu.VMEM((_NCH, _CHUNK), jnp.int32),
        pltpu.VMEM((2, _CHUNK, 128), jnp.float32),
        pltpu.VMEM((_CHUNK * _EMB,), jnp.float32),
        pltpu.SemaphoreType.DMA,
        pltpu.SemaphoreType.DMA,
    ],
    compiler_params=pltpu.CompilerParams(
        use_tc_tiling_on_sc=False, needs_layout_passes=False
    ),
)


def kernel(unk_inputs, table):
    idx = jnp.transpose(unk_inputs).reshape(_NW, _NCH, _CHUNK)
    vrow = idx >> 2
    coloff = (idx & 3) << 5
    table128 = table.reshape(_VROWS, 128)
    out = _gather(vrow, coloff, table128)
    return out.reshape(_L, _B, _EMB)


# speedup vs baseline: 1.3535x; 1.3535x over previous
import jax
import jax.numpy as jnp
from jax import lax
from jax.experimental import pallas as pl
from jax.experimental.pallas import tpu as pltpu, tpu_sc as plsc

_VOCAB = 1000000
_EMB = 32
_B = 4096
_L = 50
_NC = 2
_NS = 16
_NW = _NC * _NS
_TOTAL = _B * _L
_PER_W = _TOTAL // _NW
_CHUNK = 128
_NCH = _PER_W // _CHUNK
_CPL = _B // _CHUNK

_mesh = plsc.VectorSubcoreMesh(
    core_axis_name="c", subcore_axis_name="s", num_cores=_NC, num_subcores=_NS
)


def _gather_body(idx_hbm, table_hbm, out_hbm, idx_v, rows_v, gsem0, gsem1):
    wid = lax.axis_index("s") * _NC + lax.axis_index("c")
    pltpu.sync_copy(idx_hbm.at[pl.ds(wid * _NCH, _NCH)], idx_v)

    def _fire(j, b, sem):
        pltpu.async_copy(table_hbm.at[idx_v.at[j]], rows_v.at[b], sem)

    def _drain(b, sem):
        pltpu.make_async_copy(
            table_hbm.at[pl.ds(0, _CHUNK)], rows_v.at[b], sem
        ).wait()

    def _write(j, b):
        g = wid * _NCH + j
        l = g // _CPL
        b0 = (g % _CPL) * _CHUNK
        pltpu.sync_copy(rows_v.at[b], out_hbm.at[l, pl.ds(b0, _CHUNK)])

    _fire(0, 0, gsem0)

    @pl.loop(0, _NCH, step=2)
    def _loop(j0):
        _fire(j0 + 1, 1, gsem1)
        _drain(0, gsem0)
        _write(j0, 0)

        @pl.when(j0 + 2 < _NCH)
        def _():
            _fire(j0 + 2, 0, gsem0)

        _drain(1, gsem1)
        _write(j0 + 1, 1)


_gather = pl.kernel(
    _gather_body,
    out_type=jax.ShapeDtypeStruct((_L, _B, _EMB), jnp.float32),
    mesh=_mesh,
    scratch_types=[
        pltpu.VMEM((_NCH, _CHUNK), jnp.int32),
        pltpu.VMEM((2, _CHUNK, _EMB), jnp.float32),
        pltpu.SemaphoreType.DMA,
        pltpu.SemaphoreType.DMA,
    ],
    compiler_params=pltpu.CompilerParams(use_tc_tiling_on_sc=False),
)


def kernel(unk_inputs, table):
    idx = jnp.transpose(unk_inputs).reshape(_TOTAL // _CHUNK, _CHUNK)
    return _gather(idx, table)
